# Initial kernel scaffold; baseline (speedup 1.0000x reference)
import functools

import jax
import jax.numpy as jnp
from jax import lax
from jax.experimental import pallas as pl
from jax.experimental.pallas import tpu as pltpu
from jax.experimental.pallas import tpu_sc as plsc

N = 10000
NPAD = 10240
D = 128
H = 128
O = 64
E = 320000

NC = 2
NS = 16
NW = NC * NS

CHD = 80
ROWS_D = E // CHD
PERW_D = ROWS_D // NW

CHP = 125
ROWS_P = E // CHP
PERW_P = ROWS_P // NW

TPC = NPAD // NS

_MESH = plsc.VectorSubcoreMesh(core_axis_name="c", subcore_axis_name="s")


@functools.partial(
    pl.kernel,
    out_type=jax.ShapeDtypeStruct((NC, NPAD, 1), jnp.float32),
    mesh=_MESH,
    scratch_types=[
        pltpu.VMEM((PERW_D, CHD), jnp.int32),
        pltpu.VMEM((CHD, 1), jnp.float32),
        pltpu.VMEM_SHARED((NPAD, 1), jnp.float32),
    ],
)
def _sc_degree(dst2d, ones_col, zeros_col, deg_out, idx_v, ones_v, acc):
    cid = lax.axis_index("c")
    sid = lax.axis_index("s")
    wid = cid * NS + sid
    pltpu.sync_copy(dst2d.at[pl.ds(wid * PERW_D, PERW_D)], idx_v)
    pltpu.sync_copy(ones_col.at[pl.ds(0, CHD)], ones_v)
    pltpu.sync_copy(zeros_col.at[pl.ds(sid * TPC, TPC)],
                    acc.at[pl.ds(sid * TPC, TPC)])
    plsc.subcore_barrier()

    def body(j, _):
        pltpu.sync_copy(ones_v, acc.at[idx_v.at[j]], add=True)
        return 0

    lax.fori_loop(0, PERW_D, body, 0)
    plsc.subcore_barrier()
    pltpu.sync_copy(acc.at[pl.ds(sid * TPC, TPC)],
                    deg_out.at[cid].at[pl.ds(sid * TPC, TPC)])


@functools.partial(
    pl.kernel,
    out_type=jax.ShapeDtypeStruct((NC, NPAD, D), jnp.float32),
    mesh=_MESH,
    scratch_types=[
        pltpu.VMEM((PERW_P, CHP), jnp.int32),
        pltpu.VMEM((PERW_P, CHP), jnp.int32),
        pltpu.VMEM((CHP, D), jnp.float32),
        pltpu.VMEM_SHARED((NPAD, D), jnp.float32),
        pltpu.SemaphoreType.DMA,
    ],
)
def _sc_propagate(src2d, dst2d, u_hbm, zeros2d, s_out,
                  src_v, dst_v, rows_v, acc, sem):
    cid = lax.axis_index("c")
    sid = lax.axis_index("s")
    wid = cid * NS + sid
    pltpu.sync_copy(src2d.at[pl.ds(wid * PERW_P, PERW_P)], src_v)
    pltpu.sync_copy(dst2d.at[pl.ds(wid * PERW_P, PERW_P)], dst_v)
    pltpu.sync_copy(zeros2d.at[pl.ds(sid * TPC, TPC)],
                    acc.at[pl.ds(sid * TPC, TPC)])
    plsc.subcore_barrier()

    def body(j, _):
        pltpu.async_copy(u_hbm.at[src_v.at[j]], rows_v, sem).wait()
        pltpu.sync_copy(rows_v, acc.at[dst_v.at[j]], add=True)
        return 0

    lax.fori_loop(0, PERW_P, body, 0)
    plsc.subcore_barrier()
    pltpu.sync_copy(acc.at[pl.ds(sid * TPC, TPC)],
                    s_out.at[cid].at[pl.ds(sid * TPC, TPC)])


def _tc_prologue_body(x_ref, deg_ref, u_ref, dinv_ref):
    d = deg_ref[0] + deg_ref[1]
    dinv = lax.rsqrt(d[:N] + 1.0)
    dinv_ref[...] = dinv
    u_ref[...] = x_ref[...] * dinv


def _tc_prologue(xf, deg_parts):
    return pl.pallas_call(
        _tc_prologue_body,
        out_shape=(
            jax.ShapeDtypeStruct((N, D), jnp.float32),
            jax.ShapeDtypeStruct((N, 1), jnp.float32),
        ),
    )(xf, deg_parts)


def _tc_epilogue_body(s_ref, u_ref, dinv_ref,
                      Wz_ref, bz_ref, Lz_ref, lzb_ref,
                      Wh_ref, bh_ref, Lh_ref, lhb_ref,
                      Wout_ref, bout_ref, out_ref):
    y = (s_ref[0] + s_ref[1] + u_ref[...]) * dinv_ref[...]
    cz = jnp.dot(y, Wz_ref[...], preferred_element_type=jnp.float32) + bz_ref[...]
    z = jax.nn.sigmoid(
        jnp.dot(cz, Lz_ref[...], preferred_element_type=jnp.float32) + lzb_ref[...])
    ch = jnp.dot(y, Wh_ref[...], preferred_element_type=jnp.float32) + bh_ref[...]
    ht = jnp.tanh(
        jnp.dot(ch, Lh_ref[...], preferred_element_type=jnp.float32) + lhb_ref[...])
    out_ref[...] = (
        jnp.dot((1.0 - z) * ht, Wout_ref[...], preferred_element_type=jnp.float32)
        + bout_ref[...])


def _tc_epilogue(s_parts, u, dinv, Wz, bz, Lz1, lzb, Wh, bh, Lh1, lhb, Wout, bout):
    nblk = 10
    blk = N // nblk
    full = lambda i: (0, 0)
    return pl.pallas_call(
        _tc_epilogue_body,
        grid=(nblk,),
        in_specs=[
            pl.BlockSpec((NC, blk, D), lambda i: (0, i, 0)),
            pl.BlockSpec((blk, D), lambda i: (i, 0)),
            pl.BlockSpec((blk, 1), lambda i: (i, 0)),
            pl.BlockSpec((D, H), full),
            pl.BlockSpec((1, H), full),
            pl.BlockSpec((H, H), full),
            pl.BlockSpec((1, H), full),
            pl.BlockSpec((D, H), full),
            pl.BlockSpec((1, H), full),
            pl.BlockSpec((H, H), full),
            pl.BlockSpec((1, H), full),
            pl.BlockSpec((H, O), full),
            pl.BlockSpec((1, O), full),
        ],
        out_specs=pl.BlockSpec((blk, O), lambda i: (i, 0)),
        out_shape=jax.ShapeDtypeStruct((N, O), jnp.float32),
    )(s_parts, u, dinv, Wz, bz, Lz1, lzb, Wh, bh, Lh1, lhb, Wout, bout)


def kernel(x, edge_index, Wz, bz, Wr, br, Wh, bh,
           Lz, lzb, Lr, lrb, Lh, lhb, Wout, bout):
    xf = x.reshape(N, D)
    src = edge_index[0]
    dst = edge_index[1]

    dst2d_deg = dst.reshape(ROWS_D, CHD)
    src2d = src.reshape(ROWS_P, CHP)
    dst2d = dst.reshape(ROWS_P, CHP)

    ones_col = jnp.ones((CHD, 1), jnp.float32)
    zeros_col = jnp.zeros((NPAD, 1), jnp.float32)
    zeros2d = jnp.zeros((NPAD, D), jnp.float32)

    deg_parts = _sc_degree(dst2d_deg, ones_col, zeros_col)
    u, dinv = _tc_prologue(xf, deg_parts)
    s_parts = _sc_propagate(src2d, dst2d, u, zeros2d)

    return _tc_epilogue(
        s_parts, u, dinv,
        Wz, bz.reshape(1, H), Lz[:H], lzb.reshape(1, H),
        Wh, bh.reshape(1, H), Lh[:H], lhb.reshape(1, H),
        Wout, bout.reshape(1, O))


# speedup vs baseline: 34.8749x; 34.8749x over previous
import functools

import jax
import jax.numpy as jnp
from jax import lax
from jax.experimental import pallas as pl
from jax.experimental.pallas import tpu as pltpu
from jax.experimental.pallas import tpu_sc as plsc

N = 10000
NPAD = 10240
D = 128
H = 128
O = 64
E = 320000

NC = 2
NS = 16
NW = NC * NS

CHD = 125
ROWS_D = E // CHD
PERW_D = ROWS_D // NW

CHP = 125
ROWS_P = E // CHP
PERW_P = ROWS_P // NW

TPC = NPAD // NS

_MESH = plsc.VectorSubcoreMesh(core_axis_name="c", subcore_axis_name="s")


@functools.partial(
    pl.kernel,
    out_type=jax.ShapeDtypeStruct((NC, NPAD, 1), jnp.float32),
    mesh=_MESH,
    scratch_types=[
        pltpu.VMEM((PERW_D, CHD), jnp.int32),
        pltpu.VMEM((CHD, 1), jnp.float32),
        pltpu.VMEM_SHARED((NPAD, 1), jnp.float32),
    ],
)
def _sc_degree(dst2d, ones_col, zeros_col, deg_out, idx_v, ones_v, acc):
    cid = lax.axis_index("c")
    sid = lax.axis_index("s")
    wid = cid * NS + sid
    pltpu.sync_copy(dst2d.at[pl.ds(wid * PERW_D, PERW_D)], idx_v)
    pltpu.sync_copy(ones_col.at[pl.ds(0, CHD)], ones_v)
    pltpu.sync_copy(zeros_col.at[pl.ds(sid * TPC, TPC)],
                    acc.at[pl.ds(sid * TPC, TPC)])
    plsc.subcore_barrier()

    def body(j, _):
        pltpu.sync_copy(ones_v, acc.at[idx_v.at[j]], add=True)
        return 0

    lax.fori_loop(0, PERW_D, body, 0)
    plsc.subcore_barrier()
    pltpu.sync_copy(acc.at[pl.ds(sid * TPC, TPC)],
                    deg_out.at[cid].at[pl.ds(sid * TPC, TPC)])


@functools.partial(
    pl.kernel,
    out_type=jax.ShapeDtypeStruct((NC, NPAD, D), jnp.float32),
    mesh=_MESH,
    scratch_types=[
        pltpu.VMEM((PERW_P, CHP), jnp.int32),
        pltpu.VMEM((PERW_P, CHP), jnp.int32),
        pltpu.VMEM((CHP, D), jnp.float32),
        pltpu.VMEM_SHARED((NPAD, D), jnp.float32),
        pltpu.SemaphoreType.DMA,
    ],
)
def _sc_propagate(src2d, dst2d, u_hbm, zeros2d, s_out,
                  src_v, dst_v, rows_v, acc, sem):
    cid = lax.axis_index("c")
    sid = lax.axis_index("s")
    wid = cid * NS + sid
    pltpu.sync_copy(src2d.at[pl.ds(wid * PERW_P, PERW_P)], src_v)
    pltpu.sync_copy(dst2d.at[pl.ds(wid * PERW_P, PERW_P)], dst_v)
    pltpu.sync_copy(zeros2d.at[pl.ds(sid * TPC, TPC)],
                    acc.at[pl.ds(sid * TPC, TPC)])
    plsc.subcore_barrier()

    def body(j, _):
        pltpu.async_copy(u_hbm.at[src_v.at[j]], rows_v, sem).wait()
        pltpu.sync_copy(rows_v, acc.at[dst_v.at[j]], add=True)
        return 0

    lax.fori_loop(0, PERW_P, body, 0)
    plsc.subcore_barrier()
    pltpu.sync_copy(acc.at[pl.ds(sid * TPC, TPC)],
                    s_out.at[cid].at[pl.ds(sid * TPC, TPC)])


def _tc_prologue_body(x_ref, deg_ref, u_ref, dinv_ref):
    d = deg_ref[0] + deg_ref[1]
    dinv = lax.rsqrt(d[:N] + 1.0)
    dinv_ref[...] = dinv
    u_ref[...] = x_ref[...] * dinv


def _tc_prologue(xf, deg_parts):
    return pl.pallas_call(
        _tc_prologue_body,
        out_shape=(
            jax.ShapeDtypeStruct((N, D), jnp.float32),
            jax.ShapeDtypeStruct((N, 1), jnp.float32),
        ),
    )(xf, deg_parts)


def _tc_epilogue_body(s_ref, u_ref, dinv_ref,
                      Wz_ref, bz_ref, Lz_ref, lzb_ref,
                      Wh_ref, bh_ref, Lh_ref, lhb_ref,
                      Wout_ref, bout_ref, out_ref):
    y = (s_ref[0] + s_ref[1] + u_ref[...]) * dinv_ref[...]
    cz = jnp.dot(y, Wz_ref[...], preferred_element_type=jnp.float32) + bz_ref[...]
    z = jax.nn.sigmoid(
        jnp.dot(cz, Lz_ref[...], preferred_element_type=jnp.float32) + lzb_ref[...])
    ch = jnp.dot(y, Wh_ref[...], preferred_element_type=jnp.float32) + bh_ref[...]
    ht = jnp.tanh(
        jnp.dot(ch, Lh_ref[...], preferred_element_type=jnp.float32) + lhb_ref[...])
    out_ref[...] = (
        jnp.dot((1.0 - z) * ht, Wout_ref[...], preferred_element_type=jnp.float32)
        + bout_ref[...])


def _tc_epilogue(s_parts, u, dinv, Wz, bz, Lz1, lzb, Wh, bh, Lh1, lhb, Wout, bout):
    nblk = 10
    blk = N // nblk
    full = lambda i: (0, 0)
    return pl.pallas_call(
        _tc_epilogue_body,
        grid=(nblk,),
        in_specs=[
            pl.BlockSpec((NC, blk, D), lambda i: (0, i, 0)),
            pl.BlockSpec((blk, D), lambda i: (i, 0)),
            pl.BlockSpec((blk, 1), lambda i: (i, 0)),
            pl.BlockSpec((D, H), full),
            pl.BlockSpec((1, H), full),
            pl.BlockSpec((H, H), full),
            pl.BlockSpec((1, H), full),
            pl.BlockSpec((D, H), full),
            pl.BlockSpec((1, H), full),
            pl.BlockSpec((H, H), full),
            pl.BlockSpec((1, H), full),
            pl.BlockSpec((H, O), full),
            pl.BlockSpec((1, O), full),
        ],
        out_specs=pl.BlockSpec((blk, O), lambda i: (i, 0)),
        out_shape=jax.ShapeDtypeStruct((N, O), jnp.float32),
    )(s_parts, u, dinv, Wz, bz, Lz1, lzb, Wh, bh, Lh1, lhb, Wout, bout)


def kernel(x, edge_index, Wz, bz, Wr, br, Wh, bh,
           Lz, lzb, Lr, lrb, Lh, lhb, Wout, bout):
    xf = x.reshape(N, D)
    src = edge_index[0]
    dst = edge_index[1]

    dst2d_deg = dst.reshape(ROWS_D, CHD)
    src2d = src.reshape(ROWS_P, CHP)
    dst2d = dst.reshape(ROWS_P, CHP)

    ones_col = jnp.ones((CHD, 1), jnp.float32)
    zeros_col = jnp.zeros((NPAD, 1), jnp.float32)
    zeros2d = jnp.zeros((NPAD, D), jnp.float32)

    deg_parts = _sc_degree(dst2d_deg, ones_col, zeros_col)
    u, dinv = _tc_prologue(xf, deg_parts)
    s_parts = _sc_propagate(src2d, dst2d, u, zeros2d)

    return _tc_epilogue(
        s_parts, u, dinv,
        Wz, bz.reshape(1, H), Lz[:H], lzb.reshape(1, H),
        Wh, bh.reshape(1, H), Lh[:H], lhb.reshape(1, H),
        Wout, bout.reshape(1, O))


# speedup vs baseline: 45.2726x; 1.2981x over previous
import functools

import jax
import jax.numpy as jnp
from jax import lax
from jax.experimental import pallas as pl
from jax.experimental.pallas import tpu as pltpu
from jax.experimental.pallas import tpu_sc as plsc

N = 10000
NPAD = 10240
D = 128
H = 128
O = 64
E = 320000

NC = 2
NS = 16
NW = NC * NS

CHD = 125
ROWS_D = E // CHD
PERW_D = ROWS_D // NW

CHP = 125
ROWS_P = E // CHP
PERW_P = ROWS_P // NW

TPC = NPAD // NS

_MESH = plsc.VectorSubcoreMesh(core_axis_name="c", subcore_axis_name="s")


@functools.partial(
    pl.kernel,
    out_type=jax.ShapeDtypeStruct((NC, NPAD, 1), jnp.float32),
    mesh=_MESH,
    scratch_types=[
        pltpu.VMEM((PERW_D, CHD), jnp.int32),
        pltpu.VMEM((CHD, 1), jnp.float32),
        pltpu.VMEM_SHARED((NPAD, 1), jnp.float32),
    ],
)
def _sc_degree(dst2d, ones_col, zeros_col, deg_out, idx_v, ones_v, acc):
    cid = lax.axis_index("c")
    sid = lax.axis_index("s")
    wid = cid * NS + sid
    pltpu.sync_copy(dst2d.at[pl.ds(wid * PERW_D, PERW_D)], idx_v)
    pltpu.sync_copy(ones_col.at[pl.ds(0, CHD)], ones_v)
    pltpu.sync_copy(zeros_col.at[pl.ds(sid * TPC, TPC)],
                    acc.at[pl.ds(sid * TPC, TPC)])
    plsc.subcore_barrier()

    def body(j, _):
        pltpu.sync_copy(ones_v, acc.at[idx_v.at[j]], add=True)
        return 0

    lax.fori_loop(0, PERW_D, body, 0)
    plsc.subcore_barrier()
    pltpu.sync_copy(acc.at[pl.ds(sid * TPC, TPC)],
                    deg_out.at[cid].at[pl.ds(sid * TPC, TPC)])


NBUF = 2
NHALF = 2
PERH = PERW_P // NHALF

@functools.partial(
    pl.kernel,
    out_type=jax.ShapeDtypeStruct((NC, NPAD, D), jnp.float32),
    mesh=_MESH,
    scratch_types=(
        [pltpu.VMEM((PERH, CHP), jnp.int32),
         pltpu.VMEM((PERW_P, CHP), jnp.int32)]
        + [pltpu.VMEM((CHP, D), jnp.float32)] * NBUF
        + [pltpu.VMEM_SHARED((NPAD, D), jnp.float32)]
        + [pltpu.SemaphoreType.DMA] * NBUF
    ),
)
def _sc_propagate(src2d, dst2d, u_hbm, zeros2d, s_out, src_v, dst_v, *rest):
    bufs = rest[:NBUF]
    acc = rest[NBUF]
    sems = rest[NBUF + 1:]
    cid = lax.axis_index("c")
    sid = lax.axis_index("s")
    wid = cid * NS + sid
    pltpu.sync_copy(dst2d.at[pl.ds(wid * PERW_P, PERW_P)], dst_v)
    pltpu.sync_copy(zeros2d.at[pl.ds(sid * TPC, TPC)],
                    acc.at[pl.ds(sid * TPC, TPC)])
    plsc.subcore_barrier()

    for half in range(NHALF):
        pltpu.sync_copy(
            src2d.at[pl.ds(wid * PERW_P + half * PERH, PERH)], src_v)

        for b in range(NBUF):
            pltpu.async_copy(u_hbm.at[src_v.at[b]], bufs[b], sems[b])

        def body(gi, _, half=half):
            g = gi * NBUF
            for b in range(NBUF):
                c = g + b
                pltpu.make_async_copy(u_hbm.at[src_v.at[c]], bufs[b],
                                      sems[b]).wait()
                pltpu.sync_copy(bufs[b],
                                acc.at[dst_v.at[half * PERH + c]], add=True)
                nxt = jnp.minimum(c + NBUF, PERH - 1)
                pltpu.async_copy(u_hbm.at[src_v.at[nxt]], bufs[b], sems[b])
            return 0

        lax.fori_loop(0, PERH // NBUF, body, 0)
        for b in range(NBUF):
            pltpu.make_async_copy(u_hbm.at[src_v.at[PERH - 1]], bufs[b],
                                  sems[b]).wait()

    plsc.subcore_barrier()
    pltpu.sync_copy(acc.at[pl.ds(sid * TPC, TPC)],
                    s_out.at[cid].at[pl.ds(sid * TPC, TPC)])


def _tc_prologue_body(x_ref, deg_ref, u_ref, dinv_ref):
    d = deg_ref[0] + deg_ref[1]
    dinv = lax.rsqrt(d[:N] + 1.0)
    dinv_ref[...] = dinv
    u_ref[...] = x_ref[...] * dinv


def _tc_prologue(xf, deg_parts):
    return pl.pallas_call(
        _tc_prologue_body,
        out_shape=(
            jax.ShapeDtypeStruct((N, D), jnp.float32),
            jax.ShapeDtypeStruct((N, 1), jnp.float32),
        ),
    )(xf, deg_parts)


def _tc_epilogue_body(s_ref, u_ref, dinv_ref,
                      Wz_ref, bz_ref, Lz_ref, lzb_ref,
                      Wh_ref, bh_ref, Lh_ref, lhb_ref,
                      Wout_ref, bout_ref, out_ref):
    y = (s_ref[0] + s_ref[1] + u_ref[...]) * dinv_ref[...]
    cz = jnp.dot(y, Wz_ref[...], preferred_element_type=jnp.float32) + bz_ref[...]
    z = jax.nn.sigmoid(
        jnp.dot(cz, Lz_ref[...], preferred_element_type=jnp.float32) + lzb_ref[...])
    ch = jnp.dot(y, Wh_ref[...], preferred_element_type=jnp.float32) + bh_ref[...]
    ht = jnp.tanh(
        jnp.dot(ch, Lh_ref[...], preferred_element_type=jnp.float32) + lhb_ref[...])
    out_ref[...] = (
        jnp.dot((1.0 - z) * ht, Wout_ref[...], preferred_element_type=jnp.float32)
        + bout_ref[...])


def _tc_epilogue(s_parts, u, dinv,
                 Wz, bz, Lz1, lzb, Wh, bh, Lh1, lhb, Wout, bout):
    nblk = 10
    blk = N // nblk
    full = lambda i: (0, 0)
    return pl.pallas_call(
        _tc_epilogue_body,
        grid=(nblk,),
        in_specs=[
            pl.BlockSpec((NC, blk, D), lambda i: (0, i, 0)),
            pl.BlockSpec((blk, D), lambda i: (i, 0)),
            pl.BlockSpec((blk, 1), lambda i: (i, 0)),
            pl.BlockSpec((D, H), full),
            pl.BlockSpec((1, H), full),
            pl.BlockSpec((H, H), full),
            pl.BlockSpec((1, H), full),
            pl.BlockSpec((D, H), full),
            pl.BlockSpec((1, H), full),
            pl.BlockSpec((H, H), full),
            pl.BlockSpec((1, H), full),
            pl.BlockSpec((H, O), full),
            pl.BlockSpec((1, O), full),
        ],
        out_specs=pl.BlockSpec((blk, O), lambda i: (i, 0)),
        out_shape=jax.ShapeDtypeStruct((N, O), jnp.float32),
    )(s_parts, u, dinv, Wz, bz, Lz1, lzb, Wh, bh, Lh1, lhb, Wout, bout)


def kernel(x, edge_index, Wz, bz, Wr, br, Wh, bh,
           Lz, lzb, Lr, lrb, Lh, lhb, Wout, bout):
    xf = x.reshape(N, D)
    src = edge_index[0]
    dst = edge_index[1]

    dst2d_deg = dst.reshape(ROWS_D, CHD)
    src2d = src.reshape(ROWS_P, CHP)
    dst2d = dst.reshape(ROWS_P, CHP)

    ones_col = jnp.ones((CHD, 1), jnp.float32)
    zeros_col = jnp.zeros((NPAD, 1), jnp.float32)
    zeros2d = jnp.zeros((NPAD, D), jnp.float32)

    deg_parts = _sc_degree(dst2d_deg, ones_col, zeros_col)
    u, dinv = _tc_prologue(xf, deg_parts)
    s_parts = _sc_propagate(src2d, dst2d, u, zeros2d)

    return _tc_epilogue(
        s_parts, u, dinv,
        Wz, bz.reshape(1, H), Lz[:H], lzb.reshape(1, H),
        Wh, bh.reshape(1, H), Lh[:H], lhb.reshape(1, H),
        Wout, bout.reshape(1, O))
